# Initial kernel scaffold; baseline (speedup 1.0000x reference)
import jax
import jax.numpy as jnp
from jax.experimental import pallas as pl


def kernel(outputs, targets):
    raise NotImplementedError("write your pallas kernel here")


# speedup vs baseline: 22.2787x; 22.2787x over previous
import numpy as np
import jax
import jax.numpy as jnp
from jax.experimental import pallas as pl
from jax.experimental.pallas import tpu as pltpu

_NUM_CLASSES = 100
_BATCH = 16384
_TSW = 1.0
_ZS = list(range(64, 84))
_PAD = 128


def _tree_ranges(n):
    nodes = []

    def rec(lo, hi):
        if hi - lo <= 1:
            return
        mid = lo + (hi - lo) // 2
        nodes.append((lo, mid, hi))
        rec(lo, mid)
        rec(mid, hi)

    rec(0, n)
    return nodes


_RANGES = _tree_ranges(_NUM_CLASSES)


def _smooth_np():
    pair = {l: None for l in _ZS}
    for (lo, mid, hi) in _RANGES:
        children_ids = [lo, mid]
        inter = [i for i in _ZS if i in children_ids]
        union = [i for i in children_ids if i not in _ZS]
        for label in inter:
            if union:
                pair[label] = union[0]
    S = np.eye(_NUM_CLASSES, dtype=np.float32)
    for l, p in pair.items():
        if p is not None:
            S[l, l] = 0.5
            S[l, p] = 0.5
    return S


def _build_mats():
    S = _smooth_np()
    WL = np.zeros((_PAD, _PAD), np.float32)
    WR = np.zeros((_PAD, _PAD), np.float32)
    CL = np.zeros((_PAD, _PAD), np.float32)
    CR = np.zeros((_PAD, _PAD), np.float32)
    for n, (lo, mid, hi) in enumerate(_RANGES):
        WL[lo:mid, n] = 1.0 / (mid - lo)
        WR[mid:hi, n] = 1.0 / (hi - mid)
        CL[n, :_NUM_CLASSES] = S[:, lo:mid].sum(axis=1)
        CR[n, :_NUM_CLASSES] = S[:, mid:hi].sum(axis=1)
    return WL, WR, CL, CR


_WL, _WR, _CL, _CR = _build_mats()

_TB = 512


def _softplus(u):
    return jnp.maximum(u, 0.0) + jnp.log1p(jnp.exp(-jnp.abs(u)))


def _body(x_ref, t_ref, wl_ref, wr_ref, cl_ref, cr_ref, out_ref):
    i = pl.program_id(0)
    x = x_ref[...]
    L = jnp.dot(x, wl_ref[...], preferred_element_type=jnp.float32)
    R = jnp.dot(x, wr_ref[...], preferred_element_type=jnp.float32)
    z = L - R
    lpl = -_softplus(-z)
    lpr = -_softplus(z)
    g = jnp.dot(lpl, cl_ref[...], preferred_element_type=jnp.float32) + jnp.dot(
        lpr, cr_ref[...], preferred_element_type=jnp.float32
    )
    v = x + g * _TSW
    t = t_ref[...]
    cols = jax.lax.broadcasted_iota(jnp.int32, (_TB, _PAD), 1)
    val = jnp.sum(jnp.where(cols == t, v, 0.0), axis=1)
    xl = jnp.where(cols < _NUM_CLASSES, x, -jnp.inf)
    m = jnp.max(xl, axis=1)
    s = jnp.sum(jnp.exp(xl - m[:, None]), axis=1)
    lse = m + jnp.log(s)
    partial = jnp.sum(lse - val) * (1.0 / _BATCH)

    @pl.when(i == 0)
    def _():
        out_ref[...] = jnp.zeros((1, 1), jnp.float32)

    out_ref[...] += partial.reshape(1, 1)


def kernel(outputs, targets):
    xpad = jnp.pad(outputs, ((0, 0), (0, _PAD - _NUM_CLASSES)))
    t2 = targets.reshape(-1, 1).astype(jnp.int32)
    out = pl.pallas_call(
        _body,
        grid=(_BATCH // _TB,),
        in_specs=[
            pl.BlockSpec((_TB, _PAD), lambda i: (i, 0)),
            pl.BlockSpec((_TB, 1), lambda i: (i, 0)),
            pl.BlockSpec((_PAD, _PAD), lambda i: (0, 0)),
            pl.BlockSpec((_PAD, _PAD), lambda i: (0, 0)),
            pl.BlockSpec((_PAD, _PAD), lambda i: (0, 0)),
            pl.BlockSpec((_PAD, _PAD), lambda i: (0, 0)),
        ],
        out_specs=pl.BlockSpec((1, 1), lambda i: (0, 0)),
        out_shape=jax.ShapeDtypeStruct((1, 1), jnp.float32),
        compiler_params=pltpu.CompilerParams(
            dimension_semantics=("arbitrary",),
        ),
    )(xpad, t2, jnp.asarray(_WL), jnp.asarray(_WR), jnp.asarray(_CL), jnp.asarray(_CR))
    return out[0, 0]
